# Initial kernel scaffold; baseline (speedup 1.0000x reference)
import jax
import jax.numpy as jnp
from jax.experimental import pallas as pl


def kernel(x_bus, x_gen, ea_bb, ea_gb, ei_bb, ei_gb, W_lin_bus, b_lin_bus, W_lin_gen, b_lin_gen, W_el_bb, b_el_bb, W_el_gb, b_el_gb, het_W, het_b, ete_emb, eattr_W, att_W, msg_W, W_out_bus, b_out_bus, W_out_gen, b_out_gen):
    raise NotImplementedError("write your pallas kernel here")


# speedup vs baseline: 2.1010x; 2.1010x over previous
import functools

import jax
import jax.numpy as jnp
from jax import lax
from jax.experimental import pallas as pl
from jax.experimental.pallas import tpu as pltpu
from jax.experimental.pallas import tpu_sc as plsc

_NB, _NG = 40000, 10000
_N = _NB + _NG
_E1, _E2 = 400000, 100000
_E = _E1 + _E2
_EP = 524288
_EW = _EP // 32
_CK = 2048
_NCH = _EW // _CK
_NV = _N // 16
_CK2 = 128
_EPW = _EP // 16
_NCH2 = _EPW // _CK2
_NQ = _N // 4
_NH = 12544
_RPT = _NH // 16

_mesh = plsc.VectorSubcoreMesh(core_axis_name="c", subcore_axis_name="s")


def _mm(A, W, b, act=None, pre_act=False, bm=2048):
    M, K = A.shape
    N2 = W.shape[1]

    def body(a_ref, w_ref, b_ref, o_ref):
        a = a_ref[...]
        if pre_act:
            a = jnp.maximum(a, 0.0)
        acc = jnp.dot(a, w_ref[...], preferred_element_type=jnp.float32)
        acc = acc + b_ref[...]
        if act == "relu":
            acc = jnp.maximum(acc, 0.0)
        elif act == "leaky":
            acc = jnp.where(acc >= 0, acc, 0.2 * acc)
        elif act == "sigmoid":
            acc = jax.nn.sigmoid(acc)
        o_ref[...] = acc

    return pl.pallas_call(
        body,
        grid=(pl.cdiv(M, bm),),
        in_specs=[
            pl.BlockSpec((bm, K), lambda i: (i, 0)),
            pl.BlockSpec((K, N2), lambda i: (0, 0)),
            pl.BlockSpec((1, N2), lambda i: (0, 0)),
        ],
        out_specs=pl.BlockSpec((bm, N2), lambda i: (i, 0)),
        out_shape=jax.ShapeDtypeStruct((M, N2), jnp.float32),
    )(A, W, b.reshape(1, -1))


@functools.partial(
    pl.kernel,
    out_type=jax.ShapeDtypeStruct((_EP,), jnp.float32),
    mesh=_mesh,
    compiler_params=pltpu.CompilerParams(needs_layout_passes=False, use_tc_tiling_on_sc=False),
    scratch_types=[
        pltpu.VMEM((_N,), jnp.float32),
        pltpu.VMEM((_CK,), jnp.int32),
        pltpu.VMEM((_CK,), jnp.float32),
        pltpu.VMEM((_CK,), jnp.float32),
    ],
)
def _p1a(tab_hbm, src_hbm, b_hbm, t_hbm, tab_v, idx_v, b_v, t_v):
    c = lax.axis_index("c")
    s = lax.axis_index("s")
    wid = s * 2 + c
    pltpu.sync_copy(tab_hbm, tab_v)

    def chunk(ci, carry):
        base = wid * _EW + ci * _CK
        pltpu.sync_copy(src_hbm.at[pl.ds(base, _CK)], idx_v)
        pltpu.sync_copy(b_hbm.at[pl.ds(base, _CK)], b_v)

        def inner(j, carry2):
            sl = pl.ds(j * 16, 16)
            g = plsc.load_gather(tab_v, [idx_v[sl]])
            t_v[sl] = g + b_v[sl]
            return carry2

        lax.fori_loop(0, _CK // 16, inner, 0)
        pltpu.sync_copy(t_v, t_hbm.at[pl.ds(base, _CK)])
        return carry

    lax.fori_loop(0, _NCH, chunk, 0)


@functools.partial(
    pl.kernel,
    out_type=(
        jax.ShapeDtypeStruct((_EP,), jnp.float32),
        jax.ShapeDtypeStruct((32, _N), jnp.float32),
    ),
    mesh=_mesh,
    compiler_params=pltpu.CompilerParams(needs_layout_passes=False, use_tc_tiling_on_sc=False),
    scratch_types=[
        pltpu.VMEM((_N,), jnp.float32),
        pltpu.VMEM((_N,), jnp.float32),
        pltpu.VMEM((_CK,), jnp.int32),
        pltpu.VMEM((_CK,), jnp.float32),
        pltpu.VMEM((_CK,), jnp.float32),
    ],
)
def _p1b(tab_hbm, dst_hbm, t_hbm, ex_hbm, dpart_hbm,
         tab_v, den_v, idx_v, t_v, ex_v):
    c = lax.axis_index("c")
    s = lax.axis_index("s")
    wid = s * 2 + c
    pltpu.sync_copy(tab_hbm, tab_v)

    def zero(j, carry):
        den_v[pl.ds(j * 16, 16)] = jnp.zeros((16,), jnp.float32)
        return carry

    lax.fori_loop(0, _NV, zero, 0)

    def chunk(ci, carry):
        base = wid * _EW + ci * _CK
        pltpu.sync_copy(dst_hbm.at[pl.ds(base, _CK)], idx_v)
        pltpu.sync_copy(t_hbm.at[pl.ds(base, _CK)], t_v)

        def inner(j, carry2):
            sl = pl.ds(j * 16, 16)
            iv = idx_v[sl]
            a = plsc.load_gather(tab_v, [iv]) + t_v[sl]
            a = jnp.where(a >= 0, a, a * 0.2)
            e = jnp.exp(a)
            ex_v[sl] = e
            plsc.addupdate_scatter(den_v, [iv], e)
            return carry2

        lax.fori_loop(0, _CK // 16, inner, 0)
        pltpu.sync_copy(ex_v, ex_hbm.at[pl.ds(base, _CK)])
        return carry

    lax.fori_loop(0, _NCH, chunk, 0)
    pltpu.sync_copy(den_v, dpart_hbm.at[wid])


@functools.partial(
    pl.kernel,
    out_type=jax.ShapeDtypeStruct((4 * _NH, 64), jnp.float32),
    mesh=_mesh,
    compiler_params=pltpu.CompilerParams(needs_layout_passes=False, use_tc_tiling_on_sc=False),
    scratch_types=[
        pltpu.VMEM((_N,), jnp.float32),
        pltpu.VMEM((_CK2,), jnp.int32),
        pltpu.VMEM((_CK2,), jnp.int32),
        pltpu.VMEM((_CK2,), jnp.float32),
        pltpu.VMEM((_CK2,), jnp.float32),
        pltpu.VMEM((_CK2, 64), jnp.float32),
        pltpu.VMEM((_CK2, 64), jnp.float32),
        pltpu.VMEM((16, 64), jnp.float32),
        pltpu.VMEM_SHARED((_NH, 64), jnp.float32),
    ],
)
def _p2(den_hbm, src_hbm, dst_hbm, ex_hbm, u_hbm, v_hbm, out_hbm,
        dtab_v, src_v, dst_v, ex_v, w_v, urows, vrows, zb, out_sh):
    c = lax.axis_index("c")
    s = lax.axis_index("s")
    pltpu.sync_copy(den_hbm, dtab_v)
    for j in range(16):
        for q in range(4):
            zb[j, pl.ds(q * 16, 16)] = jnp.zeros((16,), jnp.float32)

    def zz(k, carry):
        pltpu.sync_copy(zb, out_sh.at[pl.ds(s * _RPT + k * 16, 16)])
        return carry

    def one_pass(q, lo, chunk):
        lax.fori_loop(0, _RPT // 16, zz, 0)
        plsc.subcore_barrier()
        lax.fori_loop(0, _NCH2, chunk, 0)
        plsc.subcore_barrier()
        pltpu.sync_copy(out_sh.at[pl.ds(s * _RPT, _RPT)],
                        out_hbm.at[pl.ds(q * _NH + s * _RPT, _RPT)])
        plsc.subcore_barrier()

    def make_chunk(lo):
        def chunk(ci, carry):
            base = s * _EPW + ci * _CK2
            pltpu.sync_copy(src_hbm.at[pl.ds(base, _CK2)], src_v)
            pltpu.sync_copy(dst_hbm.at[pl.ds(base, _CK2)], dst_v)
            pltpu.sync_copy(ex_hbm.at[pl.ds(base, _CK2)], ex_v)
            pltpu.sync_copy(u_hbm.at[src_v], urows)
            pltpu.sync_copy(v_hbm.at[pl.ds(base, _CK2)], vrows)

            def f1(j, carry2):
                sl = pl.ds(j * 16, 16)
                dv = dst_v[sl]
                dn = plsc.load_gather(dtab_v, [dv])
                wv = ex_v[sl] / (dn + 1e-16)
                inr = (dv >= lo) & (dv < lo + _NQ)
                w_v[sl] = jnp.where(inr, wv, 0.0)
                dst_v[sl] = jnp.where(inr, dv - lo, _NH - 1)
                return carry2

            lax.fori_loop(0, _CK2 // 16, f1, 0)

            def f2(j, carry2):
                wb = plsc.load_gather(w_v, [jnp.full((16,), j, jnp.int32)])
                for q in range(4):
                    sl = pl.ds(q * 16, 16)
                    urows[j, sl] = wb * (urows[j, sl] + vrows[j, sl])
                return carry2

            lax.fori_loop(0, _CK2, f2, 0)
            pltpu.sync_copy(urows, out_sh.at[dst_v], add=True)
            return carry
        return chunk

    for p in range(2):
        q = 2 * p + c
        one_pass(q, q * _NQ, make_chunk(q * _NQ))


def kernel(x_bus, x_gen, ea_bb, ea_gb, ei_bb, ei_gb, W_lin_bus, b_lin_bus,
           W_lin_gen, b_lin_gen, W_el_bb, b_el_bb, W_el_gb, b_el_gb,
           het_W, het_b, ete_emb, eattr_W, att_W, msg_W,
           W_out_bus, b_out_bus, W_out_gen, b_out_gen):
    pad = _EP - _E
    xb = _mm(x_bus, W_lin_bus, b_lin_bus, act="relu")
    xg = _mm(x_gen, W_lin_gen, b_lin_gen, act="relu")
    e1 = _mm(ea_bb, W_el_bb, b_el_bb, act="relu", bm=8192)
    e2 = _mm(ea_gb, W_el_gb, b_el_gb, act="relu", bm=8192)
    eattr = jnp.concatenate([e1, e2, jnp.zeros((pad, 64), jnp.float32)], 0)
    srcp = jnp.concatenate([ei_bb[0], ei_gb[0] + _NB,
                            jnp.zeros((pad,), jnp.int32)])
    dstp = jnp.concatenate([ei_bb[1], ei_gb[1],
                            jnp.zeros((pad,), jnp.int32)])
    epos = jnp.arange(_EP)
    x = jnp.concatenate([xb, xg], 0)
    for l in range(3):
        pre = l > 0
        hb = _mm(x[:_NB], het_W[l, 0], het_b[l, 0], pre_act=pre)
        hg = _mm(x[_NB:], het_W[l, 1], het_b[l, 1], pre_act=pre)
        xh = jnp.concatenate([hb, hg], 0)
        Ws = jnp.concatenate([att_W[l][0:64], att_W[l][64:128],
                              msg_W[l][0:64]], axis=1)
        S = _mm(xh, Ws, jnp.zeros((66,), jnp.float32))
        sdst = S[:, 0]
        ssrc = S[:, 1]
        u = S[:, 2:]
        eae = _mm(eattr, eattr_W[l], jnp.zeros((16,), jnp.float32),
                  act="leaky", bm=8192)
        Wv = jnp.concatenate([att_W[l][144:160], msg_W[l][64:80]], axis=1)
        ev = _mm(eae, Wv, jnp.zeros((65,), jnp.float32), bm=8192)
        ete = jnp.where(ete_emb[l] >= 0, ete_emb[l], 0.2 * ete_emb[l])
        cet = ete @ att_W[l][128:144, 0]
        b_edge = ev[:, 0] + jnp.where(
            epos < _E1, cet[0], jnp.where(epos < _E, cet[1], -1e30))
        v = ev[:, 1:]
        t = _p1a(ssrc, srcp, b_edge)
        ex, dpart = _p1b(sdst, dstp, t)
        den = dpart.sum(0)
        out_pad = _p2(den, srcp, dstp, ex, u, v)
        x = jnp.concatenate(
            [out_pad[q * _NH:q * _NH + _NQ] for q in range(4)], 0)
    bus = _mm(x[:_NB], W_out_bus, b_out_bus, act="sigmoid", pre_act=True)
    gen = _mm(x[_NB:], W_out_gen, b_out_gen, act="sigmoid", pre_act=True)
    return (bus, gen)


# speedup vs baseline: 2.5304x; 1.2044x over previous
import functools

import jax
import jax.numpy as jnp
from jax import lax
from jax.experimental import pallas as pl
from jax.experimental.pallas import tpu as pltpu
from jax.experimental.pallas import tpu_sc as plsc

_NB, _NG = 40000, 10000
_N = _NB + _NG
_E1, _E2 = 400000, 100000
_E = _E1 + _E2
_EP = 524288
_EW = _EP // 32
_CK = 2048
_NCH = _EW // _CK
_NV = _N // 16
_CK2 = 128
_EPW = _EP // 16
_NCH2 = _EPW // _CK2
_NQ = _N // 4
_NH = 12544
_RPT = _NH // 16

_mesh = plsc.VectorSubcoreMesh(core_axis_name="c", subcore_axis_name="s")


def _mm(A, W, b, act=None, pre_act=False, bm=2048):
    M, K = A.shape
    N2 = W.shape[1]

    def body(a_ref, w_ref, b_ref, o_ref):
        a = a_ref[...]
        if pre_act:
            a = jnp.maximum(a, 0.0)
        acc = jnp.dot(a, w_ref[...], preferred_element_type=jnp.float32)
        acc = acc + b_ref[...]
        if act == "relu":
            acc = jnp.maximum(acc, 0.0)
        elif act == "leaky":
            acc = jnp.where(acc >= 0, acc, 0.2 * acc)
        elif act == "sigmoid":
            acc = jax.nn.sigmoid(acc)
        o_ref[...] = acc

    return pl.pallas_call(
        body,
        grid=(pl.cdiv(M, bm),),
        in_specs=[
            pl.BlockSpec((bm, K), lambda i: (i, 0)),
            pl.BlockSpec((K, N2), lambda i: (0, 0)),
            pl.BlockSpec((1, N2), lambda i: (0, 0)),
        ],
        out_specs=pl.BlockSpec((bm, N2), lambda i: (i, 0)),
        out_shape=jax.ShapeDtypeStruct((M, N2), jnp.float32),
    )(A, W, b.reshape(1, -1))


@functools.partial(
    pl.kernel,
    out_type=jax.ShapeDtypeStruct((_EP,), jnp.float32),
    mesh=_mesh,
    compiler_params=pltpu.CompilerParams(needs_layout_passes=False, use_tc_tiling_on_sc=False),
    scratch_types=[
        pltpu.VMEM((_N,), jnp.float32),
        pltpu.VMEM((_CK,), jnp.int32),
        pltpu.VMEM((_CK,), jnp.float32),
        pltpu.VMEM((_CK,), jnp.float32),
    ],
)
def _p1a(tab_hbm, src_hbm, b_hbm, t_hbm, tab_v, idx_v, b_v, t_v):
    c = lax.axis_index("c")
    s = lax.axis_index("s")
    wid = s * 2 + c
    pltpu.sync_copy(tab_hbm, tab_v)

    def chunk(ci, carry):
        base = wid * _EW + ci * _CK
        pltpu.sync_copy(src_hbm.at[pl.ds(base, _CK)], idx_v)
        pltpu.sync_copy(b_hbm.at[pl.ds(base, _CK)], b_v)

        def inner(j, carry2):
            sl = pl.ds(j * 16, 16)
            g = plsc.load_gather(tab_v, [idx_v[sl]])
            t_v[sl] = g + b_v[sl]
            return carry2

        lax.fori_loop(0, _CK // 16, inner, 0)
        pltpu.sync_copy(t_v, t_hbm.at[pl.ds(base, _CK)])
        return carry

    lax.fori_loop(0, _NCH, chunk, 0)


@functools.partial(
    pl.kernel,
    out_type=(
        jax.ShapeDtypeStruct((_EP,), jnp.float32),
        jax.ShapeDtypeStruct((32, _N), jnp.float32),
    ),
    mesh=_mesh,
    compiler_params=pltpu.CompilerParams(needs_layout_passes=False, use_tc_tiling_on_sc=False),
    scratch_types=[
        pltpu.VMEM((_N,), jnp.float32),
        pltpu.VMEM((_N,), jnp.float32),
        pltpu.VMEM((_CK,), jnp.int32),
        pltpu.VMEM((_CK,), jnp.float32),
        pltpu.VMEM((_CK,), jnp.float32),
    ],
)
def _p1b(tab_hbm, dst_hbm, t_hbm, ex_hbm, dpart_hbm,
         tab_v, den_v, idx_v, t_v, ex_v):
    c = lax.axis_index("c")
    s = lax.axis_index("s")
    wid = s * 2 + c
    pltpu.sync_copy(tab_hbm, tab_v)

    def zero(j, carry):
        den_v[pl.ds(j * 16, 16)] = jnp.zeros((16,), jnp.float32)
        return carry

    lax.fori_loop(0, _NV, zero, 0)

    def chunk(ci, carry):
        base = wid * _EW + ci * _CK
        pltpu.sync_copy(dst_hbm.at[pl.ds(base, _CK)], idx_v)
        pltpu.sync_copy(t_hbm.at[pl.ds(base, _CK)], t_v)

        def inner(j, carry2):
            sl = pl.ds(j * 16, 16)
            iv = idx_v[sl]
            a = plsc.load_gather(tab_v, [iv]) + t_v[sl]
            a = jnp.where(a >= 0, a, a * 0.2)
            e = jnp.exp(a)
            ex_v[sl] = e
            plsc.addupdate_scatter(den_v, [iv], e)
            return carry2

        lax.fori_loop(0, _CK // 16, inner, 0)
        pltpu.sync_copy(ex_v, ex_hbm.at[pl.ds(base, _CK)])
        return carry

    lax.fori_loop(0, _NCH, chunk, 0)
    pltpu.sync_copy(den_v, dpart_hbm.at[wid])


@functools.partial(
    pl.kernel,
    out_type=jax.ShapeDtypeStruct((4 * _NH, 64), jnp.float32),
    mesh=_mesh,
    compiler_params=pltpu.CompilerParams(needs_layout_passes=False, use_tc_tiling_on_sc=False),
    scratch_types=[
        pltpu.VMEM((_N,), jnp.float32),
        pltpu.VMEM((_CK2,), jnp.int32),
        pltpu.VMEM((_CK2,), jnp.int32),
        pltpu.VMEM((_CK2,), jnp.float32),
        pltpu.VMEM((_CK2,), jnp.float32),
        pltpu.VMEM((_CK2, 64), jnp.float32),
        pltpu.VMEM((_CK2, 64), jnp.float32),
        pltpu.VMEM((16, 64), jnp.float32),
        pltpu.VMEM_SHARED((_NH, 64), jnp.float32),
        pltpu.SemaphoreType.DMA,
        pltpu.SemaphoreType.DMA,
        pltpu.SemaphoreType.DMA,
        pltpu.SemaphoreType.DMA,
        pltpu.SemaphoreType.DMA,
    ],
)
def _p2(den_hbm, src_hbm, dst_hbm, ex_hbm, u_hbm, v_hbm, out_hbm,
        dtab_v, src_v, dst_v, ex_v, w_v, urows, vrows, zb, out_sh,
        sem1, sem2, sem3, sem4, sem5):
    c = lax.axis_index("c")
    s = lax.axis_index("s")
    pltpu.sync_copy(den_hbm, dtab_v)
    for j in range(16):
        for q in range(4):
            zb[j, pl.ds(q * 16, 16)] = jnp.zeros((16,), jnp.float32)

    def zz(k, carry):
        pltpu.sync_copy(zb, out_sh.at[pl.ds(s * _RPT + k * 16, 16)])
        return carry

    def one_pass(q, lo, chunk):
        lax.fori_loop(0, _RPT // 16, zz, 0)
        plsc.subcore_barrier()
        lax.fori_loop(0, _NCH2, chunk, 0)
        plsc.subcore_barrier()
        pltpu.sync_copy(out_sh.at[pl.ds(s * _RPT, _RPT)],
                        out_hbm.at[pl.ds(q * _NH + s * _RPT, _RPT)])
        plsc.subcore_barrier()

    def make_chunk(lo):
        def chunk(ci, carry):
            base = s * _EPW + ci * _CK2
            h1 = pltpu.async_copy(src_hbm.at[pl.ds(base, _CK2)], src_v, sem1)
            h2 = pltpu.async_copy(dst_hbm.at[pl.ds(base, _CK2)], dst_v, sem2)
            h3 = pltpu.async_copy(ex_hbm.at[pl.ds(base, _CK2)], ex_v, sem3)
            h5 = pltpu.async_copy(v_hbm.at[pl.ds(base, _CK2)], vrows, sem5)
            h1.wait()
            h4 = pltpu.async_copy(u_hbm.at[src_v], urows, sem4)
            h2.wait()
            h3.wait()
            h5.wait()
            h4.wait()

            def f1(j, carry2):
                sl = pl.ds(j * 16, 16)
                dv = dst_v[sl]
                dn = plsc.load_gather(dtab_v, [dv])
                wv = ex_v[sl] / (dn + 1e-16)
                inr = (dv >= lo) & (dv < lo + _NQ)
                w_v[sl] = jnp.where(inr, wv, 0.0)
                dst_v[sl] = jnp.where(inr, dv - lo, _NH - 1)
                return carry2

            lax.fori_loop(0, _CK2 // 16, f1, 0)

            def f2(j, carry2):
                wb = plsc.load_gather(w_v, [jnp.full((16,), j, jnp.int32)])
                for q in range(4):
                    sl = pl.ds(q * 16, 16)
                    urows[j, sl] = wb * (urows[j, sl] + vrows[j, sl])
                return carry2

            lax.fori_loop(0, _CK2, f2, 0)
            pltpu.sync_copy(urows, out_sh.at[dst_v], add=True)
            return carry
        return chunk

    for p in range(2):
        q = 2 * p + c
        one_pass(q, q * _NQ, make_chunk(q * _NQ))


def kernel(x_bus, x_gen, ea_bb, ea_gb, ei_bb, ei_gb, W_lin_bus, b_lin_bus,
           W_lin_gen, b_lin_gen, W_el_bb, b_el_bb, W_el_gb, b_el_gb,
           het_W, het_b, ete_emb, eattr_W, att_W, msg_W,
           W_out_bus, b_out_bus, W_out_gen, b_out_gen):
    pad = _EP - _E
    xb = _mm(x_bus, W_lin_bus, b_lin_bus, act="relu")
    xg = _mm(x_gen, W_lin_gen, b_lin_gen, act="relu")
    e1 = _mm(ea_bb, W_el_bb, b_el_bb, act="relu", bm=8192)
    e2 = _mm(ea_gb, W_el_gb, b_el_gb, act="relu", bm=8192)
    eattr = jnp.concatenate([e1, e2, jnp.zeros((pad, 64), jnp.float32)], 0)
    srcp = jnp.concatenate([ei_bb[0], ei_gb[0] + _NB,
                            jnp.zeros((pad,), jnp.int32)])
    dstp = jnp.concatenate([ei_bb[1], ei_gb[1],
                            jnp.zeros((pad,), jnp.int32)])
    epos = jnp.arange(_EP)
    x = jnp.concatenate([xb, xg], 0)
    for l in range(3):
        pre = l > 0
        hb = _mm(x[:_NB], het_W[l, 0], het_b[l, 0], pre_act=pre)
        hg = _mm(x[_NB:], het_W[l, 1], het_b[l, 1], pre_act=pre)
        xh = jnp.concatenate([hb, hg], 0)
        Ws = jnp.concatenate([att_W[l][0:64], att_W[l][64:128],
                              msg_W[l][0:64]], axis=1)
        S = _mm(xh, Ws, jnp.zeros((66,), jnp.float32))
        sdst = S[:, 0]
        ssrc = S[:, 1]
        u = S[:, 2:]
        eae = _mm(eattr, eattr_W[l], jnp.zeros((16,), jnp.float32),
                  act="leaky", bm=8192)
        Wv = jnp.concatenate([att_W[l][144:160], msg_W[l][64:80]], axis=1)
        ev = _mm(eae, Wv, jnp.zeros((65,), jnp.float32), bm=8192)
        ete = jnp.where(ete_emb[l] >= 0, ete_emb[l], 0.2 * ete_emb[l])
        cet = ete @ att_W[l][128:144, 0]
        b_edge = ev[:, 0] + jnp.where(
            epos < _E1, cet[0], jnp.where(epos < _E, cet[1], -1e30))
        v = ev[:, 1:]
        t = _p1a(ssrc, srcp, b_edge)
        ex, dpart = _p1b(sdst, dstp, t)
        den = dpart.sum(0)
        out_pad = _p2(den, srcp, dstp, ex, u, v)
        x = jnp.concatenate(
            [out_pad[q * _NH:q * _NH + _NQ] for q in range(4)], 0)
    bus = _mm(x[:_NB], W_out_bus, b_out_bus, act="sigmoid", pre_act=True)
    gen = _mm(x[_NB:], W_out_gen, b_out_gen, act="sigmoid", pre_act=True)
    return (bus, gen)


# speedup vs baseline: 3.0886x; 1.2206x over previous
import functools

import jax
import jax.numpy as jnp
from jax import lax
from jax.experimental import pallas as pl
from jax.experimental.pallas import tpu as pltpu
from jax.experimental.pallas import tpu_sc as plsc

_NB, _NG = 40000, 10000
_N = _NB + _NG
_E1, _E2 = 400000, 100000
_E = _E1 + _E2
_EP = 524288
_EW = _EP // 32
_CK = 2048
_NCH = _EW // _CK
_NV = _N // 16
_CK2 = 128
_EPW = _EP // 16
_NCH2 = _EPW // _CK2
_NQ = _N // 4
_NH = 12544
_RPT = _NH // 16

_mesh = plsc.VectorSubcoreMesh(core_axis_name="c", subcore_axis_name="s")


def _mm(A, W, b, act=None, pre_act=False, bm=2048):
    M, K = A.shape
    N2 = W.shape[1]

    def body(a_ref, w_ref, b_ref, o_ref):
        a = a_ref[...]
        if pre_act:
            a = jnp.maximum(a, 0.0)
        acc = jnp.dot(a, w_ref[...], preferred_element_type=jnp.float32)
        acc = acc + b_ref[...]
        if act == "relu":
            acc = jnp.maximum(acc, 0.0)
        elif act == "leaky":
            acc = jnp.where(acc >= 0, acc, 0.2 * acc)
        elif act == "sigmoid":
            acc = jax.nn.sigmoid(acc)
        o_ref[...] = acc

    return pl.pallas_call(
        body,
        grid=(pl.cdiv(M, bm),),
        in_specs=[
            pl.BlockSpec((bm, K), lambda i: (i, 0)),
            pl.BlockSpec((K, N2), lambda i: (0, 0)),
            pl.BlockSpec((1, N2), lambda i: (0, 0)),
        ],
        out_specs=pl.BlockSpec((bm, N2), lambda i: (i, 0)),
        out_shape=jax.ShapeDtypeStruct((M, N2), jnp.float32),
    )(A, W, b.reshape(1, -1))


@functools.partial(
    pl.kernel,
    out_type=jax.ShapeDtypeStruct((_EP,), jnp.float32),
    mesh=_mesh,
    compiler_params=pltpu.CompilerParams(needs_layout_passes=False, use_tc_tiling_on_sc=False),
    scratch_types=[
        pltpu.VMEM((_N,), jnp.float32),
        pltpu.VMEM((_CK,), jnp.int32),
        pltpu.VMEM((_CK,), jnp.float32),
        pltpu.VMEM((_CK,), jnp.float32),
    ],
)
def _p1a(tab_hbm, src_hbm, b_hbm, t_hbm, tab_v, idx_v, b_v, t_v):
    c = lax.axis_index("c")
    s = lax.axis_index("s")
    wid = s * 2 + c
    pltpu.sync_copy(tab_hbm, tab_v)

    def chunk(ci, carry):
        base = wid * _EW + ci * _CK
        pltpu.sync_copy(src_hbm.at[pl.ds(base, _CK)], idx_v)
        pltpu.sync_copy(b_hbm.at[pl.ds(base, _CK)], b_v)

        def inner(j):
            sl = pl.ds(j * 16, 16)
            g = plsc.load_gather(tab_v, [idx_v[sl]])
            t_v[sl] = g + b_v[sl]

        plsc.parallel_loop(0, _CK // 16, 1, unroll=8)(inner)
        pltpu.sync_copy(t_v, t_hbm.at[pl.ds(base, _CK)])
        return carry

    lax.fori_loop(0, _NCH, chunk, 0)


@functools.partial(
    pl.kernel,
    out_type=(
        jax.ShapeDtypeStruct((_EP,), jnp.float32),
        jax.ShapeDtypeStruct((32, _N), jnp.float32),
    ),
    mesh=_mesh,
    compiler_params=pltpu.CompilerParams(needs_layout_passes=False, use_tc_tiling_on_sc=False),
    scratch_types=[
        pltpu.VMEM((_N,), jnp.float32),
        pltpu.VMEM((_N,), jnp.float32),
        pltpu.VMEM((_CK,), jnp.int32),
        pltpu.VMEM((_CK,), jnp.float32),
        pltpu.VMEM((_CK,), jnp.float32),
    ],
)
def _p1b(tab_hbm, dst_hbm, t_hbm, ex_hbm, dpart_hbm,
         tab_v, den_v, idx_v, t_v, ex_v):
    c = lax.axis_index("c")
    s = lax.axis_index("s")
    wid = s * 2 + c
    pltpu.sync_copy(tab_hbm, tab_v)

    def zero(j, carry):
        den_v[pl.ds(j * 16, 16)] = jnp.zeros((16,), jnp.float32)
        return carry

    lax.fori_loop(0, _NV, zero, 0)

    def chunk(ci, carry):
        base = wid * _EW + ci * _CK
        pltpu.sync_copy(dst_hbm.at[pl.ds(base, _CK)], idx_v)
        pltpu.sync_copy(t_hbm.at[pl.ds(base, _CK)], t_v)

        def inner(j, carry2):
            sl = pl.ds(j * 16, 16)
            iv = idx_v[sl]
            a = plsc.load_gather(tab_v, [iv]) + t_v[sl]
            a = jnp.where(a >= 0, a, a * 0.2)
            e = jnp.exp(a)
            ex_v[sl] = e
            plsc.addupdate_scatter(den_v, [iv], e)
            return carry2

        lax.fori_loop(0, _CK // 16, inner, 0)
        pltpu.sync_copy(ex_v, ex_hbm.at[pl.ds(base, _CK)])
        return carry

    lax.fori_loop(0, _NCH, chunk, 0)
    pltpu.sync_copy(den_v, dpart_hbm.at[wid])


@functools.partial(
    pl.kernel,
    out_type=jax.ShapeDtypeStruct((4 * _NH, 64), jnp.float32),
    mesh=_mesh,
    compiler_params=pltpu.CompilerParams(needs_layout_passes=False, use_tc_tiling_on_sc=False),
    scratch_types=[
        pltpu.VMEM((_N,), jnp.float32),
        pltpu.VMEM((_CK2,), jnp.int32),
        pltpu.VMEM((_CK2,), jnp.int32),
        pltpu.VMEM((_CK2,), jnp.float32),
        pltpu.VMEM((_CK2,), jnp.float32),
        pltpu.VMEM((_CK2, 64), jnp.float32),
        pltpu.VMEM((_CK2, 64), jnp.float32),
        pltpu.VMEM((16, 64), jnp.float32),
        pltpu.VMEM_SHARED((_NH, 64), jnp.float32),
        pltpu.SemaphoreType.DMA,
        pltpu.SemaphoreType.DMA,
        pltpu.SemaphoreType.DMA,
        pltpu.SemaphoreType.DMA,
        pltpu.SemaphoreType.DMA,
    ],
)
def _p2(den_hbm, src_hbm, dst_hbm, ex_hbm, u_hbm, v_hbm, out_hbm,
        dtab_v, src_v, dst_v, ex_v, w_v, urows, vrows, zb, out_sh,
        sem1, sem2, sem3, sem4, sem5):
    c = lax.axis_index("c")
    s = lax.axis_index("s")
    pltpu.sync_copy(den_hbm, dtab_v)
    for j in range(16):
        for q in range(4):
            zb[j, pl.ds(q * 16, 16)] = jnp.zeros((16,), jnp.float32)

    def zz(k, carry):
        pltpu.sync_copy(zb, out_sh.at[pl.ds(s * _RPT + k * 16, 16)])
        return carry

    def one_pass(q, lo, chunk):
        lax.fori_loop(0, _RPT // 16, zz, 0)
        plsc.subcore_barrier()
        lax.fori_loop(0, _NCH2, chunk, 0)
        plsc.subcore_barrier()
        pltpu.sync_copy(out_sh.at[pl.ds(s * _RPT, _RPT)],
                        out_hbm.at[pl.ds(q * _NH + s * _RPT, _RPT)])
        plsc.subcore_barrier()

    def make_chunk(lo):
        def chunk(ci, carry):
            base = s * _EPW + ci * _CK2
            h1 = pltpu.async_copy(src_hbm.at[pl.ds(base, _CK2)], src_v, sem1)
            h2 = pltpu.async_copy(dst_hbm.at[pl.ds(base, _CK2)], dst_v, sem2)
            h3 = pltpu.async_copy(ex_hbm.at[pl.ds(base, _CK2)], ex_v, sem3)
            h5 = pltpu.async_copy(v_hbm.at[pl.ds(base, _CK2)], vrows, sem5)
            h1.wait()
            h4 = pltpu.async_copy(u_hbm.at[src_v], urows, sem4)
            h2.wait()
            h3.wait()
            h5.wait()
            h4.wait()

            def f1(j):
                sl = pl.ds(j * 16, 16)
                dv = dst_v[sl]
                dn = plsc.load_gather(dtab_v, [dv])
                wv = ex_v[sl] / (dn + 1e-16)
                inr = (dv >= lo) & (dv < lo + _NQ)
                w_v[sl] = jnp.where(inr, wv, 0.0)
                dst_v[sl] = jnp.where(inr, dv - lo, _NH - 1)

            plsc.parallel_loop(0, _CK2 // 16, 1, unroll=8)(f1)

            def f2(j):
                wb = plsc.load_gather(w_v, [jnp.full((16,), j, jnp.int32)])
                for q in range(4):
                    sl = pl.ds(q * 16, 16)
                    urows[j, sl] = wb * (urows[j, sl] + vrows[j, sl])

            plsc.parallel_loop(0, _CK2, 1, unroll=8)(f2)
            pltpu.sync_copy(urows, out_sh.at[dst_v], add=True)
            return carry
        return chunk

    for p in range(2):
        q = 2 * p + c
        one_pass(q, q * _NQ, make_chunk(q * _NQ))


def kernel(x_bus, x_gen, ea_bb, ea_gb, ei_bb, ei_gb, W_lin_bus, b_lin_bus,
           W_lin_gen, b_lin_gen, W_el_bb, b_el_bb, W_el_gb, b_el_gb,
           het_W, het_b, ete_emb, eattr_W, att_W, msg_W,
           W_out_bus, b_out_bus, W_out_gen, b_out_gen):
    pad = _EP - _E
    xb = _mm(x_bus, W_lin_bus, b_lin_bus, act="relu")
    xg = _mm(x_gen, W_lin_gen, b_lin_gen, act="relu")
    e1 = _mm(ea_bb, W_el_bb, b_el_bb, act="relu", bm=8192)
    e2 = _mm(ea_gb, W_el_gb, b_el_gb, act="relu", bm=8192)
    eattr = jnp.concatenate([e1, e2, jnp.zeros((pad, 64), jnp.float32)], 0)
    srcp = jnp.concatenate([ei_bb[0], ei_gb[0] + _NB,
                            jnp.zeros((pad,), jnp.int32)])
    dstp = jnp.concatenate([ei_bb[1], ei_gb[1],
                            jnp.zeros((pad,), jnp.int32)])
    epos = jnp.arange(_EP)
    x = jnp.concatenate([xb, xg], 0)
    for l in range(3):
        pre = l > 0
        hb = _mm(x[:_NB], het_W[l, 0], het_b[l, 0], pre_act=pre)
        hg = _mm(x[_NB:], het_W[l, 1], het_b[l, 1], pre_act=pre)
        xh = jnp.concatenate([hb, hg], 0)
        Ws = jnp.concatenate([att_W[l][0:64], att_W[l][64:128],
                              msg_W[l][0:64]], axis=1)
        S = _mm(xh, Ws, jnp.zeros((66,), jnp.float32))
        sdst = S[:, 0]
        ssrc = S[:, 1]
        u = S[:, 2:]
        eae = _mm(eattr, eattr_W[l], jnp.zeros((16,), jnp.float32),
                  act="leaky", bm=8192)
        Wv = jnp.concatenate([att_W[l][144:160], msg_W[l][64:80]], axis=1)
        ev = _mm(eae, Wv, jnp.zeros((65,), jnp.float32), bm=8192)
        ete = jnp.where(ete_emb[l] >= 0, ete_emb[l], 0.2 * ete_emb[l])
        cet = ete @ att_W[l][128:144, 0]
        b_edge = ev[:, 0] + jnp.where(
            epos < _E1, cet[0], jnp.where(epos < _E, cet[1], -1e30))
        v = ev[:, 1:]
        t = _p1a(ssrc, srcp, b_edge)
        ex, dpart = _p1b(sdst, dstp, t)
        den = dpart.sum(0)
        out_pad = _p2(den, srcp, dstp, ex, u, v)
        x = jnp.concatenate(
            [out_pad[q * _NH:q * _NH + _NQ] for q in range(4)], 0)
    bus = _mm(x[:_NB], W_out_bus, b_out_bus, act="sigmoid", pre_act=True)
    gen = _mm(x[_NB:], W_out_gen, b_out_gen, act="sigmoid", pre_act=True)
    return (bus, gen)
